# baseline (speedup 1.0000x reference)
import functools

import jax
import jax.numpy as jnp
from jax import lax
from jax.experimental import pallas as pl
from jax.experimental.pallas import tpu as pltpu
from jax.experimental.pallas import tpu_sc as plsc

N = 16384
D = 768
C = 5
EPS = 1e-6
BLOCK = 4096

NC = 2
NS = 16
L = 16
NW = NC * NS
RPT = N // NW


def _sc_rank_weights(p):
    ws = []
    amax = []
    for j in range(C):
        r = jnp.zeros((L,), jnp.float32)
        for k in range(C):
            if k == j:
                continue
            beat = (p[k] >= p[j]) if k < j else (p[k] > p[j])
            r = r + jnp.where(beat, 1.0, 0.0)
        ws.append(jnp.where(r >= 0.5, 1.1 - 0.1 * r, 0.0))
        amax.append(jnp.where(r < 0.5, 1.0, 0.0))
    return ws, amax


@functools.partial(
    pl.kernel,
    mesh=plsc.VectorSubcoreMesh(core_axis_name="c", subcore_axis_name="s"),
    out_type=jax.ShapeDtypeStruct((4 * C, N), jnp.float32),
    scratch_types=[
        pltpu.VMEM((C, RPT), jnp.float32),
        pltpu.VMEM((C, RPT), jnp.float32),
        pltpu.VMEM((4 * C, RPT), jnp.float32),
    ],
)
def _sc_ranks(t1_hbm, t2_hbm, coef_hbm, t1v, t2v, cv):
    wid = lax.axis_index("s") * NC + lax.axis_index("c")
    base = wid * RPT
    pltpu.sync_copy(t1_hbm.at[:, pl.ds(base, RPT)], t1v)
    pltpu.sync_copy(t2_hbm.at[:, pl.ds(base, RPT)], t2v)

    def group(g, carry):
        sl = pl.ds(g * L, L)
        p1 = [t1v[j, sl] for j in range(C)]
        p2 = [t2v[j, sl] for j in range(C)]
        w1, a1 = _sc_rank_weights(p1)
        w2, a2 = _sc_rank_weights(p2)
        for j in range(C):
            cv[j, sl] = w1[j]
            cv[C + j, sl] = a1[j]
            cv[2 * C + j, sl] = w2[j]
            cv[3 * C + j, sl] = a2[j]
        return carry

    lax.fori_loop(0, RPT // L, group, 0)
    pltpu.sync_copy(cv, coef_hbm.at[:, pl.ds(base, RPT)])


def _tc_body(stu_ref, coef_ref, cw_ref, out_ref, cvec_ref):
    i = pl.program_id(0)
    stu = stu_ref[...]
    cw = cw_ref[...]
    coef = coef_ref[...]

    dots = lax.dot_general(cw, stu, (((1,), (1,)), ((), ())),
                           preferred_element_type=jnp.float32,
                           precision=lax.Precision.DEFAULT)
    tt = stu * (stu + 2.0 * EPS)
    ones = jnp.ones((1, D), jnp.float32)
    msum = lax.dot_general(ones, tt, (((1,), (1,)), ((), ())),
                           preferred_element_type=jnp.float32,
                           precision=lax.Precision.DEFAULT)

    @pl.when(i == 0)
    def _():
        cvec_ref[...] = (jnp.sum(cw * (cw - 2.0 * EPS), axis=1, keepdims=True)
                         + D * EPS * EPS)

    cvec = cvec_ref[...]
    d2 = msum - 2.0 * dots + cvec
    d = jnp.sqrt(jnp.maximum(d2, 0.0))
    dlo = d[0:C, :]
    dhi = d[C:2 * C, :]

    w1 = coef[0:C, :]
    a1 = coef[C:2 * C, :]
    w2 = coef[2 * C:3 * C, :]
    a2 = coef[3 * C:4 * C, :]

    da1 = jnp.sum(a2 * dhi, axis=0, keepdims=True)
    da2 = jnp.sum(a1 * dlo, axis=0, keepdims=True)
    term1 = jnp.sum(w1 * jnp.maximum(da1 - dlo + 1.0, 0.0))
    term2 = jnp.sum(w2 * jnp.maximum(da2 - dhi + 1.0, 0.0))
    part = (term1 + term2) * (1.0 / N)

    @pl.when(i == 0)
    def _():
        out_ref[0, 0] = 0.0

    out_ref[0, 0] += part


@jax.jit
def kernel(stu_emb, t1_prob, t2_prob, classifier_weight):
    t1t = t1_prob.T
    t2t = t2_prob.T
    coef = _sc_ranks(t1t, t2t)
    out = pl.pallas_call(
        _tc_body,
        grid=(N // BLOCK,),
        in_specs=[
            pl.BlockSpec((BLOCK, D), lambda i: (i, 0)),
            pl.BlockSpec((4 * C, BLOCK), lambda i: (0, i)),
            pl.BlockSpec((2 * C, D), lambda i: (0, 0)),
        ],
        out_specs=pl.BlockSpec((1, 1), lambda i: (0, 0),
                               memory_space=pltpu.SMEM),
        out_shape=jax.ShapeDtypeStruct((1, 1), jnp.float32),
        scratch_shapes=[pltpu.VMEM((2 * C, 1), jnp.float32)],
    )(stu_emb, coef, classifier_weight)
    return out[0, 0]


# speedup vs baseline: 1.1025x; 1.1025x over previous
import functools

import jax
import jax.numpy as jnp
from jax import lax
from jax.experimental import pallas as pl
from jax.experimental.pallas import tpu as pltpu
from jax.experimental.pallas import tpu_sc as plsc

N = 16384
D = 768
C = 5
EPS = 1e-6
BLOCK = 4096

NC = 2
NS = 16
L = 16
NW = NC * NS
RPT = N // NW


def _sc_rank_weights(p):
    ws = []
    amax = []
    for j in range(C):
        r = jnp.zeros((L,), jnp.float32)
        for k in range(C):
            if k == j:
                continue
            beat = (p[k] >= p[j]) if k < j else (p[k] > p[j])
            r = r + jnp.where(beat, 1.0, 0.0)
        ws.append(jnp.where(r >= 0.5, 1.1 - 0.1 * r, 0.0))
        amax.append(jnp.where(r < 0.5, 1.0, 0.0))
    return ws, amax


@functools.partial(
    pl.kernel,
    mesh=plsc.VectorSubcoreMesh(core_axis_name="c", subcore_axis_name="s"),
    out_type=jax.ShapeDtypeStruct((4 * C, N), jnp.float32),
    scratch_types=[
        pltpu.VMEM((C, RPT), jnp.float32),
        pltpu.VMEM((C, RPT), jnp.float32),
        pltpu.VMEM((4 * C, RPT), jnp.float32),
    ],
)
def _sc_ranks(t1_hbm, t2_hbm, coef_hbm, t1v, t2v, cv):
    wid = lax.axis_index("s") * NC + lax.axis_index("c")
    base = wid * RPT
    pltpu.sync_copy(t1_hbm.at[:, pl.ds(base, RPT)], t1v)
    pltpu.sync_copy(t2_hbm.at[:, pl.ds(base, RPT)], t2v)

    def group(g, carry):
        sl = pl.ds(g * L, L)
        p1 = [t1v[j, sl] for j in range(C)]
        p2 = [t2v[j, sl] for j in range(C)]
        w1, a1 = _sc_rank_weights(p1)
        w2, a2 = _sc_rank_weights(p2)
        for j in range(C):
            cv[j, sl] = w1[j]
            cv[C + j, sl] = a1[j]
            cv[2 * C + j, sl] = w2[j]
            cv[3 * C + j, sl] = a2[j]
        return carry

    lax.fori_loop(0, RPT // L, group, 0)
    pltpu.sync_copy(cv, coef_hbm.at[:, pl.ds(base, RPT)])


def _tc_dist_body(stu_ref, cw_ref, d_ref, cvec_ref):
    i = pl.program_id(0)
    stu = stu_ref[...]
    cw = cw_ref[...]

    dots = lax.dot_general(cw, stu, (((1,), (1,)), ((), ())),
                           preferred_element_type=jnp.float32,
                           precision=lax.Precision.DEFAULT)
    tt = stu * (stu + 2.0 * EPS)
    ones = jnp.ones((1, D), jnp.float32)
    msum = lax.dot_general(ones, tt, (((1,), (1,)), ((), ())),
                           preferred_element_type=jnp.float32,
                           precision=lax.Precision.DEFAULT)

    @pl.when(i == 0)
    def _():
        cvec_ref[...] = (jnp.sum(cw * (cw - 2.0 * EPS), axis=1, keepdims=True)
                         + D * EPS * EPS)

    cvec = cvec_ref[...]
    d2 = msum - 2.0 * dots + cvec
    d_ref[...] = jnp.sqrt(jnp.maximum(d2, 0.0))


def _tc_loss_body(d_ref, coef_ref, out_ref):
    i = pl.program_id(0)
    d = d_ref[...]
    coef = coef_ref[...]
    dlo = d[0:C, :]
    dhi = d[C:2 * C, :]
    w1 = coef[0:C, :]
    a1 = coef[C:2 * C, :]
    w2 = coef[2 * C:3 * C, :]
    a2 = coef[3 * C:4 * C, :]

    da1 = jnp.sum(a2 * dhi, axis=0, keepdims=True)
    da2 = jnp.sum(a1 * dlo, axis=0, keepdims=True)
    term1 = jnp.sum(w1 * jnp.maximum(da1 - dlo + 1.0, 0.0))
    term2 = jnp.sum(w2 * jnp.maximum(da2 - dhi + 1.0, 0.0))
    part = (term1 + term2) * (1.0 / N)

    @pl.when(i == 0)
    def _():
        out_ref[0, 0] = 0.0

    out_ref[0, 0] += part


BLOCK2 = 8192


@jax.jit
def kernel(stu_emb, t1_prob, t2_prob, classifier_weight):
    t1t = t1_prob.T
    t2t = t2_prob.T
    coef = _sc_ranks(t1t, t2t)
    dist = pl.pallas_call(
        _tc_dist_body,
        grid=(N // BLOCK,),
        in_specs=[
            pl.BlockSpec((BLOCK, D), lambda i: (i, 0)),
            pl.BlockSpec((2 * C, D), lambda i: (0, 0)),
        ],
        out_specs=pl.BlockSpec((2 * C, BLOCK), lambda i: (0, i)),
        out_shape=jax.ShapeDtypeStruct((2 * C, N), jnp.float32),
        scratch_shapes=[pltpu.VMEM((2 * C, 1), jnp.float32)],
    )(stu_emb, classifier_weight)
    out = pl.pallas_call(
        _tc_loss_body,
        grid=(N // BLOCK2,),
        in_specs=[
            pl.BlockSpec((2 * C, BLOCK2), lambda i: (0, i)),
            pl.BlockSpec((4 * C, BLOCK2), lambda i: (0, i)),
        ],
        out_specs=pl.BlockSpec((1, 1), lambda i: (0, 0),
                               memory_space=pltpu.SMEM),
        out_shape=jax.ShapeDtypeStruct((1, 1), jnp.float32),
    )(dist, coef)
    return out[0, 0]


# speedup vs baseline: 1.1035x; 1.0009x over previous
import functools

import jax
import jax.numpy as jnp
from jax import lax
from jax.experimental import pallas as pl
from jax.experimental.pallas import tpu as pltpu
from jax.experimental.pallas import tpu_sc as plsc

N = 16384
D = 768
C = 5
EPS = 1e-6
BLOCK = 4096
BLOCK2 = 8192

NC = 2
NS = 16
L = 16
NW = NC * NS
RPT = N // NW


def _sc_rank_weights(p):
    ws = []
    amax = []
    for j in range(C):
        r = jnp.zeros((L,), jnp.float32)
        for k in range(C):
            if k == j:
                continue
            beat = (p[k] >= p[j]) if k < j else (p[k] > p[j])
            r = r + jnp.where(beat, 1.0, 0.0)
        ws.append(jnp.where(r >= 0.5, 1.1 - 0.1 * r, 0.0))
        amax.append(jnp.where(r < 0.5, 1.0, 0.0))
    return ws, amax


@functools.partial(
    pl.kernel,
    mesh=plsc.VectorSubcoreMesh(core_axis_name="c", subcore_axis_name="s"),
    out_type=jax.ShapeDtypeStruct((4 * C, N), jnp.float32),
    scratch_types=[
        pltpu.VMEM((C, RPT), jnp.float32),
        pltpu.VMEM((C, RPT), jnp.float32),
        pltpu.VMEM((4 * C, RPT), jnp.float32),
    ],
)
def _sc_ranks(t1_hbm, t2_hbm, coef_hbm, t1v, t2v, cv):
    wid = lax.axis_index("s") * NC + lax.axis_index("c")
    base = wid * RPT
    pltpu.sync_copy(t1_hbm.at[:, pl.ds(base, RPT)], t1v)
    pltpu.sync_copy(t2_hbm.at[:, pl.ds(base, RPT)], t2v)

    def group(g, carry):
        sl = pl.ds(g * L, L)
        p1 = [t1v[j, sl] for j in range(C)]
        p2 = [t2v[j, sl] for j in range(C)]
        w1, a1 = _sc_rank_weights(p1)
        w2, a2 = _sc_rank_weights(p2)
        for j in range(C):
            cv[j, sl] = w1[j]
            cv[C + j, sl] = a1[j]
            cv[2 * C + j, sl] = w2[j]
            cv[3 * C + j, sl] = a2[j]
        return carry

    lax.fori_loop(0, RPT // L, group, 0)
    pltpu.sync_copy(cv, coef_hbm.at[:, pl.ds(base, RPT)])


def _tc_dist_body(stu_ref, cw_ref, d_ref, cvec_ref):
    i = pl.program_id(0)
    stu = stu_ref[...]
    cw = cw_ref[...]

    dots = lax.dot_general(cw, stu, (((1,), (1,)), ((), ())),
                           preferred_element_type=jnp.float32,
                           precision=lax.Precision.DEFAULT)
    tt = stu * (stu + 2.0 * EPS)
    ones = jnp.ones((1, D), jnp.float32)
    msum = lax.dot_general(ones, tt, (((1,), (1,)), ((), ())),
                           preferred_element_type=jnp.float32,
                           precision=lax.Precision.DEFAULT)

    @pl.when(i == 0)
    def _():
        cvec_ref[...] = (jnp.sum(cw * (cw - 2.0 * EPS), axis=1, keepdims=True)
                         + D * EPS * EPS)

    cvec = cvec_ref[...]
    d2 = msum - 2.0 * dots + cvec
    d_ref[...] = jnp.sqrt(jnp.maximum(d2, 0.0))


def _tc_loss_body(d_ref, coef_ref, out_ref):
    i = pl.program_id(0)
    d = d_ref[...]
    coef = coef_ref[...]
    dlo = d[0:C, :]
    dhi = d[C:2 * C, :]
    w1 = coef[0:C, :]
    a1 = coef[C:2 * C, :]
    w2 = coef[2 * C:3 * C, :]
    a2 = coef[3 * C:4 * C, :]

    da1 = jnp.sum(a2 * dhi, axis=0, keepdims=True)
    da2 = jnp.sum(a1 * dlo, axis=0, keepdims=True)
    term1 = jnp.sum(w1 * jnp.maximum(da1 - dlo + 1.0, 0.0))
    term2 = jnp.sum(w2 * jnp.maximum(da2 - dhi + 1.0, 0.0))
    part = (term1 + term2) * (1.0 / N)

    @pl.when(i == 0)
    def _():
        out_ref[0, 0] = 0.0

    out_ref[0, 0] += part


@jax.jit
def kernel(stu_emb, t1_prob, t2_prob, classifier_weight):
    t1t = t1_prob.T
    t2t = t2_prob.T
    coef = _sc_ranks(t1t, t2t)
    dist = pl.pallas_call(
        _tc_dist_body,
        grid=(N // BLOCK,),
        in_specs=[
            pl.BlockSpec((BLOCK, D), lambda i: (i, 0)),
            pl.BlockSpec((2 * C, D), lambda i: (0, 0)),
        ],
        out_specs=pl.BlockSpec((2 * C, BLOCK), lambda i: (0, i)),
        out_shape=jax.ShapeDtypeStruct((2 * C, N), jnp.float32),
        scratch_shapes=[pltpu.VMEM((2 * C, 1), jnp.float32)],
    )(stu_emb, classifier_weight)
    out = pl.pallas_call(
        _tc_loss_body,
        grid=(N // BLOCK2,),
        in_specs=[
            pl.BlockSpec((2 * C, BLOCK2), lambda i: (0, i)),
            pl.BlockSpec((4 * C, BLOCK2), lambda i: (0, i)),
        ],
        out_specs=pl.BlockSpec((1, 1), lambda i: (0, 0),
                               memory_space=pltpu.SMEM),
        out_shape=jax.ShapeDtypeStruct((1, 1), jnp.float32),
    )(dist, coef)
    return out[0, 0]
